# baseline (speedup 1.0000x reference)
import jax
import jax.numpy as jnp
from jax import lax
from jax.experimental import pallas as pl
from jax.experimental.pallas import tpu as pltpu
from jax.experimental.pallas import tpu_sc as plsc

LANES = 16
NC, NS = 2, 16
NW = NC * NS

B, V, L, D = 8, 8, 4096, 128
SEGS = B * V
SEGS_PER_W = SEGS // NW

L_SC = 1536
TC_A = 512
TC_B = L - L_SC - TC_A

CHUNK = 256
NCHUNK = L_SC // CHUNK
ROW_UNROLL = 4
DV = D // LANES


def _sc_body(x_hbm, ta_hbm, out_hbm, buf, outv, tav, sem):
    wid = lax.axis_index("s") * NC + lax.axis_index("c")
    base_seg = wid * SEGS_PER_W

    for s in range(SEGS_PER_W):
        seg = base_seg + s

        def start(g):
            return pltpu.async_copy(
                x_hbm.at[seg, pl.ds(g * CHUNK, CHUNK)],
                buf.at[pl.ds(lax.rem(g, 2) * CHUNK, CHUNK)],
                sem,
            )

        start(0)
        start(1)

        def chunk_body(g, acc):
            pltpu.make_async_copy(
                x_hbm.at[seg, pl.ds(0, CHUNK)],
                buf.at[pl.ds(0, CHUNK)],
                sem,
            ).wait()
            base = lax.rem(g, 2) * CHUNK

            def row_body(i, a):
                r = base + i * ROW_UNROLL
                out = list(a)
                for k in range(ROW_UNROLL):
                    for j in range(DV):
                        out[j] = out[j] + buf[r + k, pl.ds(j * LANES, LANES)]
                return tuple(out)

            acc = lax.fori_loop(0, CHUNK // ROW_UNROLL, row_body, acc)

            @pl.when(g + 2 < NCHUNK)
            def _():
                start(g + 2)

            return acc

        acc = tuple(jnp.zeros((LANES,), jnp.float32) for _ in range(DV))
        acc = lax.fori_loop(0, NCHUNK, chunk_body, acc)
        for j in range(DV):
            outv[s, pl.ds(j * LANES, LANES)] = acc[j] * (1.0 / float(D))

    pltpu.sync_copy(ta_hbm.at[pl.ds(base_seg, SEGS_PER_W)], tav)
    for s in range(SEGS_PER_W):
        for j in range(DV):
            sl = pl.ds(j * LANES, LANES)
            outv[s, sl] = outv[s, sl] + tav[s, sl]

    pltpu.sync_copy(outv, out_hbm.at[pl.ds(base_seg, SEGS_PER_W)])


def _make_tc_body(row0, nrows, ring):
    def tc_body(x_hbm, o_ref, bufs, sems):
        def copy(seg, slot):
            return pltpu.make_async_copy(
                x_hbm.at[seg, pl.ds(row0, nrows)], bufs.at[slot],
                sems.at[slot],
            )

        for k in range(ring):
            copy(k, k).start()

        def body(p, _):
            for k in range(ring):
                seg = ring * p + k
                copy(seg, k).wait()
                acc = jnp.sum(bufs[k], axis=0, keepdims=True)
                o_ref[pl.ds(seg, 1), :] = acc * (1.0 / float(D))

                @pl.when(seg + ring < SEGS)
                def _():
                    copy(seg + ring, k).start()

            return 0

        lax.fori_loop(0, SEGS // ring, body, 0)

    return tc_body


def _tc_call(row0, nrows, ring, x):
    return pl.pallas_call(
        _make_tc_body(row0, nrows, ring),
        in_specs=[pl.BlockSpec(memory_space=pl.ANY)],
        out_specs=pl.BlockSpec(memory_space=pltpu.VMEM),
        out_shape=jax.ShapeDtypeStruct((SEGS, D), jnp.float32),
        scratch_shapes=[
            pltpu.VMEM((ring, nrows, D), jnp.float32),
            pltpu.SemaphoreType.DMA((ring,)),
        ],
    )(x)


@jax.jit
def kernel(X):
    xf = X.reshape(SEGS, L, D)

    tc_a = _tc_call(L_SC, TC_A, 4, xf)

    sc_part = pl.kernel(
        _sc_body,
        out_type=jax.ShapeDtypeStruct((SEGS, D), jnp.float32),
        mesh=plsc.VectorSubcoreMesh(core_axis_name="c", subcore_axis_name="s"),
        scratch_types=[
            pltpu.VMEM((2 * CHUNK, D), jnp.float32),
            pltpu.VMEM((SEGS_PER_W, D), jnp.float32),
            pltpu.VMEM((SEGS_PER_W, D), jnp.float32),
            pltpu.SemaphoreType.DMA,
        ],
    )(xf, tc_a)

    tc_b = _tc_call(L_SC + TC_A, TC_B, 4, xf)

    ret = sc_part + tc_b
    ret = jnp.where(jnp.isnan(ret), jnp.zeros_like(ret), ret)
    return ret.reshape(B, V, D)


# speedup vs baseline: 1.1811x; 1.1811x over previous
import jax
import jax.numpy as jnp
from jax import lax
from jax.experimental import pallas as pl
from jax.experimental.pallas import tpu as pltpu
from jax.experimental.pallas import tpu_sc as plsc

LANES = 16
NC, NS = 2, 16
NW = NC * NS

B, V, L, D = 8, 8, 4096, 128
SEGS = B * V
SEGS_PER_W = SEGS // NW

L_SC = 1536
TC_B = L - L_SC

CHUNK = 384
NCHUNK = L_SC // CHUNK
ROW_UNROLL = 4
DV = D // LANES


def _sc_body(x_hbm, out_hbm, buf, outv, sem):
    wid = lax.axis_index("s") * NC + lax.axis_index("c")
    base_seg = wid * SEGS_PER_W

    for s in range(SEGS_PER_W):
        seg = base_seg + s

        def start(g):
            return pltpu.async_copy(
                x_hbm.at[seg, pl.ds(g * CHUNK, CHUNK)],
                buf.at[pl.ds(lax.rem(g, 2) * CHUNK, CHUNK)],
                sem,
            )

        start(0)
        start(1)

        def chunk_body(g, acc):
            pltpu.make_async_copy(
                x_hbm.at[seg, pl.ds(0, CHUNK)],
                buf.at[pl.ds(0, CHUNK)],
                sem,
            ).wait()
            base = lax.rem(g, 2) * CHUNK

            def row_body(i, a):
                r = base + i * ROW_UNROLL
                out = list(a)
                for k in range(ROW_UNROLL):
                    for j in range(DV):
                        out[j] = out[j] + buf[r + k, pl.ds(j * LANES, LANES)]
                return tuple(out)

            acc = lax.fori_loop(0, CHUNK // ROW_UNROLL, row_body, acc)

            @pl.when(g + 2 < NCHUNK)
            def _():
                start(g + 2)

            return acc

        acc = tuple(jnp.zeros((LANES,), jnp.float32) for _ in range(DV))
        acc = lax.fori_loop(0, NCHUNK, chunk_body, acc)
        for j in range(DV):
            outv[s, pl.ds(j * LANES, LANES)] = acc[j] * (1.0 / float(D))

    pltpu.sync_copy(outv, out_hbm.at[pl.ds(base_seg, SEGS_PER_W)])


def _make_tc_body(row0, nrows, ring):
    def tc_body(x_hbm, o_ref, bufs, sems):
        def copy(seg, slot):
            return pltpu.make_async_copy(
                x_hbm.at[seg, pl.ds(row0, nrows)], bufs.at[slot],
                sems.at[slot],
            )

        for k in range(ring):
            copy(k, k).start()

        def body(p, _):
            for k in range(ring):
                seg = ring * p + k
                copy(seg, k).wait()
                acc = jnp.sum(bufs[k], axis=0, keepdims=True)
                o_ref[pl.ds(seg, 1), :] = acc * (1.0 / float(D))

                @pl.when(seg + ring < SEGS)
                def _():
                    copy(seg + ring, k).start()

            return 0

        lax.fori_loop(0, SEGS // ring, body, 0)

    return tc_body


def _tc_call(row0, nrows, ring, x):
    return pl.pallas_call(
        _make_tc_body(row0, nrows, ring),
        in_specs=[pl.BlockSpec(memory_space=pl.ANY)],
        out_specs=pl.BlockSpec(memory_space=pltpu.VMEM),
        out_shape=jax.ShapeDtypeStruct((SEGS, D), jnp.float32),
        scratch_shapes=[
            pltpu.VMEM((ring, nrows, D), jnp.float32),
            pltpu.SemaphoreType.DMA((ring,)),
        ],
    )(x)


@jax.jit
def kernel(X):
    xf = X.reshape(SEGS, L, D)

    sc_part = pl.kernel(
        _sc_body,
        out_type=jax.ShapeDtypeStruct((SEGS, D), jnp.float32),
        mesh=plsc.VectorSubcoreMesh(core_axis_name="c", subcore_axis_name="s"),
        scratch_types=[
            pltpu.VMEM((2 * CHUNK, D), jnp.float32),
            pltpu.VMEM((SEGS_PER_W, D), jnp.float32),
            pltpu.SemaphoreType.DMA,
        ],
    )(xf)

    tc_b = _tc_call(L_SC, TC_B, 4, xf)

    ret = sc_part + tc_b
    ret = jnp.where(jnp.isnan(ret), jnp.zeros_like(ret), ret)
    return ret.reshape(B, V, D)
